# Initial kernel scaffold; baseline (speedup 1.0000x reference)
import jax
import jax.numpy as jnp
from jax.experimental import pallas as pl


def kernel(dag_summaries, obs_ptr, W1, b1, W2, b2, W3, b3):
    raise NotImplementedError("write your pallas kernel here")


# speedup vs baseline: 36.4406x; 36.4406x over previous
import functools

import jax
import jax.numpy as jnp
from jax import lax
from jax.experimental import pallas as pl
from jax.experimental.pallas import tpu as pltpu
from jax.experimental.pallas import tpu_sc as plsc


def _stage1_packed_prefix(dag, W1, b1r, W2p, b2r, consts, R):
    N, D = dag.shape
    H1 = W1.shape[1]
    G = N // R
    Rp = R // 8
    tri, Call, S, St = consts

    def body(x_ref, w1_ref, b1_ref, w2_ref, b2_ref,
             tri_ref, call_ref, st_ref, p_ref, carry_ref):
        g = pl.program_id(0)

        @pl.when(g == 0)
        def _():
            carry_ref[...] = jnp.zeros_like(carry_ref)

        x = x_ref[...]
        h1 = jnp.maximum(
            jnp.dot(x, w1_ref[...], preferred_element_type=jnp.float32)
            + b1_ref[...], 0.0)
        h2 = jnp.maximum(
            jnp.dot(h1, w2_ref[...], preferred_element_type=jnp.float32)
            + b2_ref[...], 0.0)
        cm = jnp.dot(call_ref[...], h2, preferred_element_type=jnp.float32)
        w = jnp.concatenate([cm[k * Rp:(k + 1) * Rp, :] for k in range(8)],
                            axis=1)
        gs = cm[8 * Rp:9 * Rp, :]
        ge = jnp.dot(tri_ref[...], gs, preferred_element_type=jnp.float32,
                     precision=lax.Precision.HIGHEST)
        geb = jnp.dot(ge, st_ref[...], preferred_element_type=jnp.float32,
                      precision=lax.Precision.HIGHEST)
        carry = carry_ref[...]
        p_ref[...] = w + geb + carry
        cs = jnp.sum(gs, axis=0, keepdims=True)
        carry_ref[...] = carry + jnp.dot(
            cs, st_ref[...], preferred_element_type=jnp.float32,
            precision=lax.Precision.HIGHEST)

    return pl.pallas_call(
        body,
        grid=(G + 1,),
        in_specs=[
            pl.BlockSpec((R, D), lambda g: (jnp.minimum(g, G - 1), 0)),
            pl.BlockSpec((D, H1), lambda g: (0, 0)),
            pl.BlockSpec((1, H1), lambda g: (0, 0)),
            pl.BlockSpec((H1, 16), lambda g: (0, 0)),
            pl.BlockSpec((1, 16), lambda g: (0, 0)),
            pl.BlockSpec((Rp, Rp), lambda g: (0, 0)),
            pl.BlockSpec((9 * Rp, R), lambda g: (0, 0)),
            pl.BlockSpec((16, 128), lambda g: (0, 0)),
        ],
        out_specs=pl.BlockSpec((Rp, 128), lambda g: (g, 0)),
        out_shape=jax.ShapeDtypeStruct((N // 8 + Rp, 128), jnp.float32),
        scratch_shapes=[pltpu.VMEM((1, 128), jnp.float32)],
    )(dag, W1, b1r, W2p, b2r, tri, Call, St)


def _make_consts(Rp):
    R = Rp * 8
    m = lax.broadcasted_iota(jnp.int32, (9 * Rp, R), 0)
    r = lax.broadcasted_iota(jnp.int32, (9 * Rp, R), 1)
    Call = ((r // 8 == m % Rp) & (r % 8 < m // Rp)).astype(jnp.float32)
    r2 = lax.broadcasted_iota(jnp.int32, (128, 128), 0)
    c2 = lax.broadcasted_iota(jnp.int32, (128, 128), 1)
    S = (r2 % 16 == c2).astype(jnp.float32)[:, :16]
    St = (r2[:16] == c2[:16] % 16).astype(jnp.float32)
    rg = lax.broadcasted_iota(jnp.int32, (Rp, Rp), 0)
    cg = lax.broadcasted_iota(jnp.int32, (Rp, Rp), 1)
    tri = (cg < rg).astype(jnp.float32)
    return tri, Call, S, St


def _stage2_gather(Pp, idx_pad, b_per_w):
    Bpad = idx_pad.shape[0]
    nseg16 = b_per_w // 16
    mesh = plsc.VectorSubcoreMesh(core_axis_name="c", subcore_axis_name="s")
    info = plsc.get_sparse_core_info()
    NC = info.num_cores

    @functools.partial(
        pl.kernel,
        mesh=mesh,
        out_type=jax.ShapeDtypeStruct((Bpad, 128), jnp.float32),
        scratch_types=[
            pltpu.VMEM((b_per_w,), jnp.int32),
            pltpu.VMEM((b_per_w, 128), jnp.float32),
            pltpu.SemaphoreType.DMA,
        ],
    )
    def gather_k(p_hbm, idx_hbm, out_hbm, idx_v, rows_v, sem):
        wid = lax.axis_index("s") * NC + lax.axis_index("c")
        base = wid * b_per_w
        pltpu.sync_copy(idx_hbm.at[pl.ds(base, b_per_w)], idx_v)
        for i in range(nseg16):
            sl = pl.ds(i * 16, 16)
            idx_v[sl] = lax.shift_right_logical(idx_v[sl], 3)
        pltpu.async_copy(p_hbm.at[idx_v], rows_v, sem).wait()
        pltpu.sync_copy(rows_v, out_hbm.at[pl.ds(base, b_per_w)])

    return gather_k(Pp, idx_pad)


def _stage3_output(Gp, ptr_i, S, W3p, b3r, Bseg):
    D = W3p.shape[1]
    Bp1 = Bseg + 1

    def body(g_ref, pi_ref, s_ref, w3_ref, b3_ref, o_ref):
        gp = g_ref[...][:Bp1, :]
        pi = pi_ref[...]
        rem = jnp.bitwise_and(pi, 7)
        win = lax.broadcasted_iota(jnp.int32, (Bp1, 128), 1) // 16
        masked = jnp.where(win == rem, gp, 0.0)
        ext = jnp.dot(masked, s_ref[...],
                      preferred_element_type=jnp.float32,
                      precision=lax.Precision.HIGHEST)
        d = ext[1:, :] - ext[:Bseg, :]
        cnt = (pi[1:, :] - pi[:Bseg, :]).astype(jnp.float32)
        o_ref[...] = (
            jnp.dot(d, w3_ref[...], preferred_element_type=jnp.float32,
                    precision=lax.Precision.HIGHEST)
            + cnt * b3_ref[...])

    return pl.pallas_call(
        body,
        out_shape=jax.ShapeDtypeStruct((Bseg, D), jnp.float32),
    )(Gp, ptr_i, S, W3p, b3r)


def kernel(dag_summaries, obs_ptr, W1, b1, W2, b2, W3, b3):
    N, D = dag_summaries.shape
    H1 = W1.shape[1]
    H2 = W2.shape[1]
    Bseg = obs_ptr.shape[0] - 1
    R = 256

    ptr = obs_ptr.astype(jnp.int32)

    W2p = jnp.zeros((H1, 16), jnp.float32).at[:, :H2].set(W2)
    b2p = jnp.zeros((1, 16), jnp.float32).at[0, :H2].set(b2)
    W3p = jnp.zeros((16, D), jnp.float32).at[:H2, :].set(W3)
    b1r = b1.reshape(1, H1)
    b3r = b3.reshape(1, D)

    consts = _make_consts(R // 8)
    Pp = _stage1_packed_prefix(dag_summaries, W1, b1r, W2p, b2p, consts, R)

    info = plsc.get_sparse_core_info()
    NW = info.num_cores * info.num_subcores
    chunk = 16 * NW
    Bpad = ((Bseg + 1 + chunk - 1) // chunk) * chunk
    idx_pad = jnp.zeros((Bpad,), jnp.int32).at[:Bseg + 1].set(ptr)
    Gp = _stage2_gather(Pp, idx_pad, Bpad // NW)

    ptr_i = ptr.reshape(Bseg + 1, 1)
    return _stage3_output(Gp, ptr_i, consts[2], W3p, b3r, Bseg)


# speedup vs baseline: 61.0100x; 1.6742x over previous
import functools

import jax
import jax.numpy as jnp
from jax import lax
from jax.experimental import pallas as pl
from jax.experimental.pallas import tpu as pltpu
from jax.experimental.pallas import tpu_sc as plsc


def _stage1_packed_prefix(dag, W1, b1r, W2p, b2r, consts, R):
    N, D = dag.shape
    H1 = W1.shape[1]
    G = N // R
    Rp = R // 8
    tri, Call, S, St = consts

    def body(x_ref, w1_ref, b1_ref, w2_ref, b2_ref,
             tri_ref, call_ref, st_ref, p_ref, carry_ref):
        g = pl.program_id(0)

        @pl.when(g == 0)
        def _():
            carry_ref[...] = jnp.zeros_like(carry_ref)

        x = x_ref[...]
        h1 = jnp.maximum(
            jnp.dot(x, w1_ref[...], preferred_element_type=jnp.float32)
            + b1_ref[...], 0.0)
        h2 = jnp.maximum(
            jnp.dot(h1, w2_ref[...], preferred_element_type=jnp.float32)
            + b2_ref[...], 0.0)
        cm = jnp.dot(call_ref[...], h2, preferred_element_type=jnp.float32)
        w = jnp.concatenate([cm[k * Rp:(k + 1) * Rp, :] for k in range(8)],
                            axis=1)
        gs = cm[8 * Rp:9 * Rp, :]
        ge = jnp.dot(tri_ref[...], gs, preferred_element_type=jnp.float32,
                     precision=lax.Precision.HIGHEST)
        geb = jnp.dot(ge, st_ref[...], preferred_element_type=jnp.float32,
                      precision=lax.Precision.HIGHEST)
        carry = carry_ref[...]
        p_ref[...] = w + geb + carry
        cs = jnp.sum(gs, axis=0, keepdims=True)
        carry_ref[...] = carry + jnp.dot(
            cs, st_ref[...], preferred_element_type=jnp.float32,
            precision=lax.Precision.HIGHEST)

    return pl.pallas_call(
        body,
        grid=(G + 1,),
        in_specs=[
            pl.BlockSpec((R, D), lambda g: (jnp.minimum(g, G - 1), 0)),
            pl.BlockSpec((D, H1), lambda g: (0, 0)),
            pl.BlockSpec((1, H1), lambda g: (0, 0)),
            pl.BlockSpec((H1, 16), lambda g: (0, 0)),
            pl.BlockSpec((1, 16), lambda g: (0, 0)),
            pl.BlockSpec((Rp, Rp), lambda g: (0, 0)),
            pl.BlockSpec((9 * Rp, R), lambda g: (0, 0)),
            pl.BlockSpec((16, 128), lambda g: (0, 0)),
        ],
        out_specs=pl.BlockSpec((Rp, 128), lambda g: (g, 0)),
        out_shape=jax.ShapeDtypeStruct((N // 8 + Rp, 128), jnp.float32),
        scratch_shapes=[pltpu.VMEM((1, 128), jnp.float32)],
    )(dag, W1, b1r, W2p, b2r, tri, Call, St)


def _make_consts(Rp):
    R = Rp * 8
    m = lax.broadcasted_iota(jnp.int32, (9 * Rp, R), 0)
    r = lax.broadcasted_iota(jnp.int32, (9 * Rp, R), 1)
    Call = ((r // 8 == m % Rp) & (r % 8 < m // Rp)).astype(jnp.float32)
    r2 = lax.broadcasted_iota(jnp.int32, (128, 128), 0)
    c2 = lax.broadcasted_iota(jnp.int32, (128, 128), 1)
    S = (r2 % 16 == c2).astype(jnp.float32)[:, :16]
    St = (r2[:16] == c2[:16] % 16).astype(jnp.float32)
    rg = lax.broadcasted_iota(jnp.int32, (Rp, Rp), 0)
    cg = lax.broadcasted_iota(jnp.int32, (Rp, Rp), 1)
    tri = (cg < rg).astype(jnp.float32)
    return tri, Call, S, St


def _stage2_gather(Pp, idx_pad, b_per_w):
    Bpad = idx_pad.shape[0]
    nseg16 = b_per_w // 16
    mesh = plsc.VectorSubcoreMesh(core_axis_name="c", subcore_axis_name="s")
    info = plsc.get_sparse_core_info()
    NC = info.num_cores

    @functools.partial(
        pl.kernel,
        mesh=mesh,
        out_type=jax.ShapeDtypeStruct((Bpad, 128), jnp.float32),
        scratch_types=[
            pltpu.VMEM((b_per_w,), jnp.int32),
            pltpu.VMEM((b_per_w, 128), jnp.float32),
            pltpu.SemaphoreType.DMA,
        ],
    )
    def gather_k(p_hbm, idx_hbm, out_hbm, idx_v, rows_v, sem):
        wid = lax.axis_index("s") * NC + lax.axis_index("c")
        base = wid * b_per_w
        pltpu.sync_copy(idx_hbm.at[pl.ds(base, b_per_w)], idx_v)
        for i in range(nseg16):
            sl = pl.ds(i * 16, 16)
            idx_v[sl] = lax.shift_right_logical(idx_v[sl], 3)
        pltpu.async_copy(p_hbm.at[idx_v], rows_v, sem).wait()
        pltpu.sync_copy(rows_v, out_hbm.at[pl.ds(base, b_per_w)])

    return gather_k(Pp, idx_pad)


def _stage3_output(Gp, ptr_i, S, W3p, b3r, Bseg):
    D = W3p.shape[1]
    Bp1 = Bseg + 1

    def body(g_ref, pi_ref, s_ref, w3_ref, b3_ref, o_ref):
        gp = g_ref[...][:Bp1, :]
        pi = pi_ref[...]
        rem = jnp.bitwise_and(pi, 7)
        win = lax.broadcasted_iota(jnp.int32, (Bp1, 128), 1) // 16
        masked = jnp.where(win == rem, gp, 0.0)
        ext = jnp.dot(masked, s_ref[...],
                      preferred_element_type=jnp.float32,
                      precision=lax.Precision.HIGHEST)
        d = ext[1:, :] - ext[:Bseg, :]
        cnt = (pi[1:, :] - pi[:Bseg, :]).astype(jnp.float32)
        o_ref[...] = (
            jnp.dot(d, w3_ref[...], preferred_element_type=jnp.float32,
                    precision=lax.Precision.HIGHEST)
            + cnt * b3_ref[...])

    return pl.pallas_call(
        body,
        out_shape=jax.ShapeDtypeStruct((Bseg, D), jnp.float32),
    )(Gp, ptr_i, S, W3p, b3r)


def kernel(dag_summaries, obs_ptr, W1, b1, W2, b2, W3, b3):
    N, D = dag_summaries.shape
    H1 = W1.shape[1]
    H2 = W2.shape[1]
    Bseg = obs_ptr.shape[0] - 1
    R = 512

    ptr = obs_ptr.astype(jnp.int32)

    W2p = jnp.zeros((H1, 16), jnp.float32).at[:, :H2].set(W2)
    b2p = jnp.zeros((1, 16), jnp.float32).at[0, :H2].set(b2)
    W3p = jnp.zeros((16, D), jnp.float32).at[:H2, :].set(W3)
    b1r = b1.reshape(1, H1)
    b3r = b3.reshape(1, D)

    consts = _make_consts(R // 8)
    Pp = _stage1_packed_prefix(dag_summaries, W1, b1r, W2p, b2p, consts, R)

    info = plsc.get_sparse_core_info()
    NW = info.num_cores * info.num_subcores
    chunk = 16 * NW
    Bpad = ((Bseg + 1 + chunk - 1) // chunk) * chunk
    idx_pad = jnp.zeros((Bpad,), jnp.int32).at[:Bseg + 1].set(ptr)
    Gp = _stage2_gather(Pp, idx_pad, Bpad // NW)

    ptr_i = ptr.reshape(Bseg + 1, 1)
    return _stage3_output(Gp, ptr_i, consts[2], W3p, b3r, Bseg)


# speedup vs baseline: 75.1791x; 1.2322x over previous
import functools

import jax
import jax.numpy as jnp
from jax import lax
from jax.experimental import pallas as pl
from jax.experimental.pallas import tpu as pltpu
from jax.experimental.pallas import tpu_sc as plsc


def _stage1_packed_prefix(dag, W1, b1r, W2p, b2r, consts, R):
    N, D = dag.shape
    H1 = W1.shape[1]
    G = N // R
    Rp = R // 8
    tri, Call, S, St = consts

    def body(x_ref, w1_ref, b1_ref, w2_ref, b2_ref,
             tri_ref, call_ref, st_ref, p_ref, carry_ref):
        g = pl.program_id(0)

        @pl.when(g == 0)
        def _():
            carry_ref[...] = jnp.zeros_like(carry_ref)

        x = x_ref[...]
        h1 = jnp.maximum(
            jnp.dot(x, w1_ref[...], preferred_element_type=jnp.float32)
            + b1_ref[...], 0.0)
        h2 = jnp.maximum(
            jnp.dot(h1, w2_ref[...], preferred_element_type=jnp.float32)
            + b2_ref[...], 0.0)
        cm = jnp.dot(call_ref[...], h2, preferred_element_type=jnp.float32)
        w = jnp.concatenate([cm[k * Rp:(k + 1) * Rp, :] for k in range(8)],
                            axis=1)
        gs = cm[8 * Rp:9 * Rp, :]
        ge = jnp.dot(tri_ref[...], gs, preferred_element_type=jnp.float32,
                     precision=lax.Precision.HIGHEST)
        geb = jnp.dot(ge, st_ref[...], preferred_element_type=jnp.float32,
                      precision=lax.Precision.HIGHEST)
        carry = carry_ref[...]
        p_ref[...] = w + geb + carry
        cs = jnp.sum(gs, axis=0, keepdims=True)
        carry_ref[...] = carry + jnp.dot(
            cs, st_ref[...], preferred_element_type=jnp.float32,
            precision=lax.Precision.HIGHEST)

    return pl.pallas_call(
        body,
        grid=(G + 1,),
        in_specs=[
            pl.BlockSpec((R, D), lambda g: (jnp.minimum(g, G - 1), 0)),
            pl.BlockSpec((D, H1), lambda g: (0, 0)),
            pl.BlockSpec((1, H1), lambda g: (0, 0)),
            pl.BlockSpec((H1, 16), lambda g: (0, 0)),
            pl.BlockSpec((1, 16), lambda g: (0, 0)),
            pl.BlockSpec((Rp, Rp), lambda g: (0, 0)),
            pl.BlockSpec((9 * Rp, R), lambda g: (0, 0)),
            pl.BlockSpec((16, 128), lambda g: (0, 0)),
        ],
        out_specs=pl.BlockSpec((Rp, 128), lambda g: (g, 0)),
        out_shape=jax.ShapeDtypeStruct((N // 8 + Rp, 128), jnp.float32),
        scratch_shapes=[pltpu.VMEM((1, 128), jnp.float32)],
    )(dag, W1, b1r, W2p, b2r, tri, Call, St)


def _make_consts(Rp):
    R = Rp * 8
    m = lax.broadcasted_iota(jnp.int32, (9 * Rp, R), 0)
    r = lax.broadcasted_iota(jnp.int32, (9 * Rp, R), 1)
    Call = ((r // 8 == m % Rp) & (r % 8 < m // Rp)).astype(jnp.float32)
    r2 = lax.broadcasted_iota(jnp.int32, (128, 128), 0)
    c2 = lax.broadcasted_iota(jnp.int32, (128, 128), 1)
    S = (r2 % 16 == c2).astype(jnp.float32)[:, :16]
    St = (r2[:16] == c2[:16] % 16).astype(jnp.float32)
    rg = lax.broadcasted_iota(jnp.int32, (Rp, Rp), 0)
    cg = lax.broadcasted_iota(jnp.int32, (Rp, Rp), 1)
    tri = (cg < rg).astype(jnp.float32)
    return tri, Call, S, St


def _stage2_gather(Pp, idx_pad, b_per_w):
    Bpad = idx_pad.shape[0]
    nseg16 = b_per_w // 16
    mesh = plsc.VectorSubcoreMesh(core_axis_name="c", subcore_axis_name="s")
    info = plsc.get_sparse_core_info()
    NC = info.num_cores

    @functools.partial(
        pl.kernel,
        mesh=mesh,
        out_type=jax.ShapeDtypeStruct((Bpad, 128), jnp.float32),
        scratch_types=[
            pltpu.VMEM((b_per_w,), jnp.int32),
            pltpu.VMEM((b_per_w, 128), jnp.float32),
            pltpu.SemaphoreType.DMA,
        ],
    )
    def gather_k(p_hbm, idx_hbm, out_hbm, idx_v, rows_v, sem):
        wid = lax.axis_index("s") * NC + lax.axis_index("c")
        base = wid * b_per_w
        pltpu.sync_copy(idx_hbm.at[pl.ds(base, b_per_w)], idx_v)
        for i in range(nseg16):
            sl = pl.ds(i * 16, 16)
            idx_v[sl] = lax.shift_right_logical(idx_v[sl], 3)
        pltpu.async_copy(p_hbm.at[idx_v], rows_v, sem).wait()
        pltpu.sync_copy(rows_v, out_hbm.at[pl.ds(base, b_per_w)])

    return gather_k(Pp, idx_pad)


def _stage3_output(Gp, ptr_i, S, W3p, b3r, Bseg):
    D = W3p.shape[1]
    Bp1 = Bseg + 1

    def body(g_ref, pi_ref, s_ref, w3_ref, b3_ref, o_ref):
        gp = g_ref[...][:Bp1, :]
        pi = pi_ref[...]
        rem = jnp.bitwise_and(pi, 7)
        win = lax.broadcasted_iota(jnp.int32, (Bp1, 128), 1) // 16
        masked = jnp.where(win == rem, gp, 0.0)
        ext = jnp.dot(masked, s_ref[...],
                      preferred_element_type=jnp.float32,
                      precision=lax.Precision.HIGHEST)
        d = ext[1:, :] - ext[:Bseg, :]
        cnt = (pi[1:, :] - pi[:Bseg, :]).astype(jnp.float32)
        o_ref[...] = (
            jnp.dot(d, w3_ref[...], preferred_element_type=jnp.float32,
                    precision=lax.Precision.HIGHEST)
            + cnt * b3_ref[...])

    return pl.pallas_call(
        body,
        out_shape=jax.ShapeDtypeStruct((Bseg, D), jnp.float32),
    )(Gp, ptr_i, S, W3p, b3r)


def kernel(dag_summaries, obs_ptr, W1, b1, W2, b2, W3, b3):
    N, D = dag_summaries.shape
    H1 = W1.shape[1]
    H2 = W2.shape[1]
    Bseg = obs_ptr.shape[0] - 1
    R = 1024

    ptr = obs_ptr.astype(jnp.int32)

    W2p = jnp.zeros((H1, 16), jnp.float32).at[:, :H2].set(W2)
    b2p = jnp.zeros((1, 16), jnp.float32).at[0, :H2].set(b2)
    W3p = jnp.zeros((16, D), jnp.float32).at[:H2, :].set(W3)
    b1r = b1.reshape(1, H1)
    b3r = b3.reshape(1, D)

    consts = _make_consts(R // 8)
    Pp = _stage1_packed_prefix(dag_summaries, W1, b1r, W2p, b2p, consts, R)

    info = plsc.get_sparse_core_info()
    NW = info.num_cores * info.num_subcores
    chunk = 16 * NW
    Bpad = ((Bseg + 1 + chunk - 1) // chunk) * chunk
    idx_pad = jnp.zeros((Bpad,), jnp.int32).at[:Bseg + 1].set(ptr)
    Gp = _stage2_gather(Pp, idx_pad, Bpad // NW)

    ptr_i = ptr.reshape(Bseg + 1, 1)
    return _stage3_output(Gp, ptr_i, consts[2], W3p, b3r, Bseg)
